# baseline (speedup 1.0000x reference)
import functools

import jax
from jax import lax
from jax.experimental import pallas as pl
from jax.experimental.pallas import tpu as pltpu
from jax.experimental.pallas import tpu_sc as plsc

_CH = 32


@functools.cache
def _make_sc_broadcast(B, S, H, dtype):
    info = plsc.get_sparse_core_info()
    num_cores, num_subcores = info.num_cores, info.num_subcores
    num_workers = num_cores * num_subcores
    rows_w = S // num_workers
    n_chunks = rows_w // _CH
    mesh = plsc.VectorSubcoreMesh(core_axis_name="c", subcore_axis_name="s")

    @functools.partial(
        pl.kernel,
        out_type=jax.ShapeDtypeStruct((B, S, H), dtype),
        mesh=mesh,
        scratch_types=[
            pltpu.VMEM((_CH, H), dtype),
            pltpu.VMEM((_CH, H), dtype),
            pltpu.SemaphoreType.DMA,
            pltpu.SemaphoreType.DMA,
        ],
    )
    def sc_broadcast(table_hbm, out_hbm, buf0, buf1, rsem, wsem):
        wid = lax.axis_index("s") * num_cores + lax.axis_index("c")
        base = wid * rows_w
        bufs = (buf0, buf1)
        rcp = pltpu.async_copy(table_hbm.at[pl.ds(base, _CH)], buf0, rsem)
        for i in range(n_chunks):
            nxt = None
            if i + 1 < n_chunks:
                nxt = pltpu.async_copy(
                    table_hbm.at[pl.ds(base + (i + 1) * _CH, _CH)],
                    bufs[(i + 1) % 2],
                    rsem,
                )
            rcp.wait()
            buf = bufs[i % 2]
            r0 = base + i * _CH
            wcps = [
                pltpu.async_copy(buf, out_hbm.at[b, pl.ds(r0, _CH)], wsem)
                for b in range(B)
            ]
            for w in wcps:
                w.wait()
            rcp = nxt

    return sc_broadcast


def kernel(x, table):
    B, S = x.shape
    M, H = table.shape
    return _make_sc_broadcast(B, S, H, table.dtype)(table)


# speedup vs baseline: 1.0038x; 1.0038x over previous
import functools

import jax
from jax import lax
from jax.experimental import pallas as pl
from jax.experimental.pallas import tpu as pltpu
from jax.experimental.pallas import tpu_sc as plsc

_CH = 32


@functools.cache
def _make_sc_broadcast(B, S, H, dtype):
    info = plsc.get_sparse_core_info()
    num_cores, num_subcores = info.num_cores, info.num_subcores
    num_workers = num_cores * num_subcores
    rows_w = S // num_workers
    n_chunks = rows_w // _CH
    mesh = plsc.VectorSubcoreMesh(core_axis_name="c", subcore_axis_name="s")

    @functools.partial(
        pl.kernel,
        out_type=jax.ShapeDtypeStruct((B, S, H), dtype),
        mesh=mesh,
        scratch_types=[
            pltpu.VMEM((_CH, H), dtype),
            pltpu.VMEM((_CH, H), dtype),
            pltpu.VMEM((_CH, H), dtype),
            pltpu.SemaphoreType.DMA,
            pltpu.SemaphoreType.DMA,
        ],
    )
    def sc_broadcast(table_hbm, out_hbm, buf0, buf1, buf2, rsem, wsem):
        wid = lax.axis_index("s") * num_cores + lax.axis_index("c")
        base = wid * rows_w
        bufs = (buf0, buf1, buf2)
        nbuf = len(bufs)
        rcps = {}
        for i in range(min(nbuf - 1, n_chunks)):
            rcps[i] = pltpu.async_copy(
                table_hbm.at[pl.ds(base + i * _CH, _CH)], bufs[i % nbuf], rsem
            )
        pending = None
        for i in range(n_chunks):
            rcps.pop(i).wait()
            buf = bufs[i % nbuf]
            r0 = base + i * _CH
            wcps = [
                pltpu.async_copy(buf, out_hbm.at[b, pl.ds(r0, _CH)], wsem)
                for b in range(B)
            ]
            if pending is not None:
                for w in pending:
                    w.wait()
            if i + nbuf - 1 < n_chunks:
                j = i + nbuf - 1
                rcps[j] = pltpu.async_copy(
                    table_hbm.at[pl.ds(base + j * _CH, _CH)], bufs[j % nbuf], rsem
                )
            pending = wcps
        for w in pending:
            w.wait()

    return sc_broadcast


def kernel(x, table):
    B, S = x.shape
    M, H = table.shape
    return _make_sc_broadcast(B, S, H, table.dtype)(table)
